# Initial kernel scaffold; baseline (speedup 1.0000x reference)
import functools

import jax
import jax.numpy as jnp
from jax import lax
from jax.experimental import pallas as pl
from jax.experimental.pallas import tpu as pltpu
from jax.experimental.pallas import tpu_sc as plsc

N_NODES = 10000
D_FEAT = 128
D_EDGE = 16

NC = 2
NS = 16
NW = NC * NS

CHUNK = 256
SUB = 128
ROWS_PER_TILE = N_NODES // NS


def _sc_aggregate(features, src2d, dst2d, phys2d, edges_per_tile):
    n_chunks = edges_per_tile // CHUNK
    idx_rows_per_chunk = CHUNK // 128
    phys_rows_per_chunk = CHUNK * D_EDGE // 128

    mesh = plsc.VectorSubcoreMesh(core_axis_name="c", subcore_axis_name="s")

    @functools.partial(
        pl.kernel,
        mesh=mesh,
        out_type=jax.ShapeDtypeStruct((NC, N_NODES, D_FEAT), jnp.float32),
        scratch_types=[
            pltpu.VMEM((CHUNK, D_FEAT), jnp.float32),
            pltpu.VMEM((idx_rows_per_chunk, 128), jnp.int32),
            pltpu.VMEM((idx_rows_per_chunk, 128), jnp.int32),
            pltpu.VMEM((phys_rows_per_chunk, 128), jnp.float32),
            pltpu.VMEM_SHARED((N_NODES, D_FEAT), jnp.float32),
            pltpu.SemaphoreType.DMA,
        ],
    )
    def k(features_hbm, src_hbm, dst_hbm, phys_hbm, out_hbm,
          rows_v, src_v, dst_v, phys_v, acc_sh, sem):
        c = lax.axis_index("c")
        s = lax.axis_index("s")
        wid = s * NC + c

        zeros16 = jnp.zeros((16,), jnp.float32)

        def zero_body(r, _):
            for g in range(D_FEAT // 16):
                rows_v[r, pl.ds(g * 16, 16)] = zeros16
            return 0
        lax.fori_loop(0, CHUNK, zero_body, 0)

        base_row = s * ROWS_PER_TILE
        done = 0
        while done < ROWS_PER_TILE:
            n = min(CHUNK, ROWS_PER_TILE - done)
            pltpu.sync_copy(rows_v.at[pl.ds(0, n)], acc_sh.at[pl.ds(base_row + done, n)])
            done += n
        plsc.subcore_barrier()

        edge_base = wid * edges_per_tile

        def chunk_body(ci, _):
            idx_row = (edge_base // 128) + ci * idx_rows_per_chunk
            phys_row = (edge_base * D_EDGE // 128) + ci * phys_rows_per_chunk
            pltpu.sync_copy(src_hbm.at[pl.ds(idx_row, idx_rows_per_chunk)], src_v)
            pltpu.sync_copy(dst_hbm.at[pl.ds(idx_row, idx_rows_per_chunk)], dst_v)
            pltpu.sync_copy(phys_hbm.at[pl.ds(phys_row, phys_rows_per_chunk)], phys_v)

            cps = []
            for j in range(CHUNK // SUB):
                cps.append(pltpu.async_copy(
                    features_hbm.at[src_v.at[j]],
                    rows_v.at[pl.ds(j * SUB, SUB)],
                    sem))
            for cp in cps:
                cp.wait()

            def scale_body(r, _):
                for e in range(8):
                    w = jnp.sum(phys_v[r, pl.ds(e * 16, 16)])
                    row = r * 8 + e
                    for g in range(D_FEAT // 16):
                        sl = pl.ds(g * 16, 16)
                        rows_v[row, sl] = rows_v[row, sl] * w
                return 0
            lax.fori_loop(0, phys_rows_per_chunk, scale_body, 0)

            for j in range(CHUNK // SUB):
                pltpu.sync_copy(rows_v.at[pl.ds(j * SUB, SUB)],
                                acc_sh.at[dst_v.at[j]], add=True)
            return 0

        lax.fori_loop(0, n_chunks, chunk_body, 0)
        plsc.subcore_barrier()

        pltpu.sync_copy(acc_sh.at[pl.ds(base_row, ROWS_PER_TILE)],
                        out_hbm.at[c, pl.ds(base_row, ROWS_PER_TILE)])

    return k(features, src2d, dst2d, phys2d)


def _combine_partials(partials):
    blk = 1000

    def add_k(p_ref, o_ref):
        o_ref[...] = p_ref[0] + p_ref[1]

    return pl.pallas_call(
        add_k,
        grid=(N_NODES // blk,),
        in_specs=[pl.BlockSpec((NC, blk, D_FEAT), lambda i: (0, i, 0))],
        out_specs=pl.BlockSpec((blk, D_FEAT), lambda i: (i, 0)),
        out_shape=jax.ShapeDtypeStruct((N_NODES, D_FEAT), jnp.float32),
    )(partials)


@jax.jit
def kernel(features, adjacency_list, physics_features):
    n_edges = adjacency_list.shape[1]
    edges_per_tile = -(-n_edges // (NW * CHUNK)) * CHUNK
    e_pad = edges_per_tile * NW
    pad = e_pad - n_edges

    src = adjacency_list[0].astype(jnp.int32)
    dst = adjacency_list[1].astype(jnp.int32)
    phys = physics_features.astype(jnp.float32)
    if pad:
        src = jnp.concatenate([src, jnp.zeros((pad,), jnp.int32)])
        dst = jnp.concatenate([dst, jnp.zeros((pad,), jnp.int32)])
        phys = jnp.concatenate([phys, jnp.zeros((pad, D_EDGE), jnp.float32)])

    src2d = src.reshape(e_pad // 128, 128)
    dst2d = dst.reshape(e_pad // 128, 128)
    phys2d = phys.reshape(e_pad * D_EDGE // 128, 128)

    partials = _sc_aggregate(features, src2d, dst2d, phys2d, edges_per_tile)
    return _combine_partials(partials)


# speedup vs baseline: 2.2709x; 2.2709x over previous
import functools

import jax
import jax.numpy as jnp
from jax import lax
from jax.experimental import pallas as pl
from jax.experimental.pallas import tpu as pltpu
from jax.experimental.pallas import tpu_sc as plsc

N_NODES = 10000
N_PAD = 10240
D_FEAT = 128
D_EDGE = 16

NC = 2
NS = 16
NW = NC * NS

SUPER = 1024
CHUNK = 256
SUB = 128
ROWS_PER_TILE = N_PAD // NS


def _edge_weights(phys2d):
    rows = phys2d.shape[0]
    blk = 4096
    grid = -(-rows // blk)

    def wk(p_ref, s_ref, o_ref):
        o_ref[...] = jnp.dot(p_ref[...], s_ref[...],
                             preferred_element_type=jnp.float32)

    s_mat = jnp.repeat(jnp.eye(8, dtype=jnp.float32), D_EDGE, axis=0)
    return pl.pallas_call(
        wk,
        grid=(grid,),
        in_specs=[
            pl.BlockSpec((blk, 128), lambda i: (i, 0)),
            pl.BlockSpec((128, 8), lambda i: (0, 0)),
        ],
        out_specs=pl.BlockSpec((blk, 8), lambda i: (i, 0)),
        out_shape=jax.ShapeDtypeStruct((rows, 8), jnp.float32),
    )(phys2d, s_mat)


def _sc_aggregate(features, src2d, dst2d, w2d, edges_per_tile):
    n_super = edges_per_tile // SUPER
    idx_rows = SUPER // 128

    mesh = plsc.VectorSubcoreMesh(core_axis_name="c", subcore_axis_name="s")

    @functools.partial(
        pl.kernel,
        mesh=mesh,
        out_type=jax.ShapeDtypeStruct((NC, N_PAD, D_FEAT), jnp.float32),
        scratch_types=[
            pltpu.VMEM((CHUNK, D_FEAT), jnp.float32),
            pltpu.VMEM((idx_rows, 128), jnp.int32),
            pltpu.VMEM((idx_rows, 128), jnp.int32),
            pltpu.VMEM((idx_rows, 128), jnp.float32),
            pltpu.VMEM_SHARED((N_PAD, D_FEAT), jnp.float32),
            pltpu.SemaphoreType.DMA,
        ],
    )
    def k(features_hbm, src_hbm, dst_hbm, w_hbm, out_hbm,
          rows_v, src_v, dst_v, w_v, acc_sh, sem):
        c = lax.axis_index("c")
        s = lax.axis_index("s")
        wid = s * NC + c

        zeros16 = jnp.zeros((16,), jnp.float32)

        def zero_body(r, _):
            for g in range(D_FEAT // 16):
                rows_v[r, pl.ds(g * 16, 16)] = zeros16
            return 0
        lax.fori_loop(0, CHUNK, zero_body, 0)

        base_row = pl.multiple_of(s * ROWS_PER_TILE, 8)
        done = 0
        while done < ROWS_PER_TILE:
            n = min(CHUNK, ROWS_PER_TILE - done)
            pltpu.sync_copy(rows_v.at[pl.ds(0, n)], acc_sh.at[pl.ds(base_row + done, n)])
            done += n
        plsc.subcore_barrier()

        edge_base = wid * edges_per_tile

        def super_body(si, _):
            idx_row0 = pl.multiple_of((edge_base // 128) + si * idx_rows, 8)
            pltpu.sync_copy(src_hbm.at[pl.ds(idx_row0, idx_rows)], src_v)
            pltpu.sync_copy(dst_hbm.at[pl.ds(idx_row0, idx_rows)], dst_v)
            pltpu.sync_copy(w_hbm.at[pl.ds(idx_row0, idx_rows)], w_v)

            for h in range(SUPER // CHUNK):
                cps = []
                for j in range(CHUNK // SUB):
                    cps.append(pltpu.async_copy(
                        features_hbm.at[src_v.at[h * 2 + j]],
                        rows_v.at[pl.ds(j * SUB, SUB)],
                        sem))
                for cp in cps:
                    cp.wait()

                for hh in range(CHUNK // SUB):
                    def scale_body(gi, _, hh=hh):
                        off = pl.multiple_of(gi * 16, 16)
                        w16 = w_v[h * 2 + hh, pl.ds(off, 16)]
                        for e in range(16):
                            w = w16[e]
                            row = hh * SUB + off + e
                            for g in range(D_FEAT // 16):
                                sl = pl.ds(g * 16, 16)
                                rows_v[row, sl] = rows_v[row, sl] * w
                        return 0
                    lax.fori_loop(0, SUB // 16, scale_body, 0)

                for j in range(CHUNK // SUB):
                    pltpu.sync_copy(rows_v.at[pl.ds(j * SUB, SUB)],
                                    acc_sh.at[dst_v.at[h * 2 + j]], add=True)
            return 0

        lax.fori_loop(0, n_super, super_body, 0)
        plsc.subcore_barrier()

        pltpu.sync_copy(acc_sh.at[pl.ds(base_row, ROWS_PER_TILE)],
                        out_hbm.at[c, pl.ds(base_row, ROWS_PER_TILE)])

    return k(features, src2d, dst2d, w2d)


def _combine_partials(partials):
    blk = 1000

    def add_k(p_ref, o_ref):
        o_ref[...] = p_ref[0] + p_ref[1]

    return pl.pallas_call(
        add_k,
        grid=(N_NODES // blk,),
        in_specs=[pl.BlockSpec((NC, blk, D_FEAT), lambda i: (0, i, 0))],
        out_specs=pl.BlockSpec((blk, D_FEAT), lambda i: (i, 0)),
        out_shape=jax.ShapeDtypeStruct((N_NODES, D_FEAT), jnp.float32),
    )(partials)


@jax.jit
def kernel(features, adjacency_list, physics_features):
    n_edges = adjacency_list.shape[1]
    edges_per_tile = -(-n_edges // (NW * SUPER)) * SUPER
    e_pad = edges_per_tile * NW
    pad = e_pad - n_edges

    src = adjacency_list[0].astype(jnp.int32)
    dst = adjacency_list[1].astype(jnp.int32)
    phys = physics_features.astype(jnp.float32)
    if pad:
        src = jnp.concatenate([src, jnp.zeros((pad,), jnp.int32)])
        dst = jnp.concatenate([dst, jnp.zeros((pad,), jnp.int32)])
        phys = jnp.concatenate([phys, jnp.zeros((pad, D_EDGE), jnp.float32)])

    src2d = src.reshape(e_pad // 128, 128)
    dst2d = dst.reshape(e_pad // 128, 128)
    phys2d = phys.reshape(e_pad * D_EDGE // 128, 128)

    w2d = _edge_weights(phys2d).reshape(e_pad // 128, 128)

    partials = _sc_aggregate(features, src2d, dst2d, w2d, edges_per_tile)
    return _combine_partials(partials)


# speedup vs baseline: 2.8434x; 1.2521x over previous
import functools

import jax
import jax.numpy as jnp
from jax import lax
from jax.experimental import pallas as pl
from jax.experimental.pallas import tpu as pltpu
from jax.experimental.pallas import tpu_sc as plsc

N_NODES = 10000
N_HALF = 5120
TRASH_ROWS = 128
ACC_ROWS = N_HALF + TRASH_ROWS
D_FEAT = 128
D_EDGE = 16

NC = 2
NS = 16

CHUNK = 128
GROUP = 16
ROWS_PER_TILE = N_HALF // NS


def _sc_aggregate(features, src2d, dst2d, phys, n_edges, edges_per_tile):
    n_chunks = edges_per_tile // CHUNK
    n_groups = n_chunks // GROUP

    mesh = plsc.VectorSubcoreMesh(core_axis_name="c", subcore_axis_name="s")

    @functools.partial(
        pl.kernel,
        mesh=mesh,
        out_type=jax.ShapeDtypeStruct((NC, N_HALF, D_FEAT), jnp.float32),
        scratch_types=[
            pltpu.VMEM((2, CHUNK, D_FEAT), jnp.float32),
            pltpu.VMEM((2, CHUNK, D_EDGE), jnp.float32),
            pltpu.VMEM((GROUP, 128), jnp.int32),
            pltpu.VMEM((GROUP, 128), jnp.int32),
            pltpu.VMEM((2, 128), jnp.int32),
            pltpu.SemaphoreType.DMA,
            pltpu.SemaphoreType.DMA,
            pltpu.SemaphoreType.DMA,
            pltpu.SemaphoreType.DMA,
            pltpu.VMEM_SHARED((ACC_ROWS, D_FEAT), jnp.float32),
        ],
    )
    def k(feat_hbm, src_hbm, dst_hbm, phys_hbm, out_hbm,
          gbuf, pbuf, src_v, dst_v, adj_v,
          gsem0, gsem1, ssem0, ssem1, acc_sh):
        gsem = (gsem0, gsem1)
        ssem = (ssem0, ssem1)
        c = lax.axis_index("c")
        s = lax.axis_index("s")
        lo = c * N_HALF
        edge_base = s * edges_per_tile
        idx_base = pl.multiple_of(edge_base // 128, 8)
        nv = jnp.clip((n_edges - edge_base) // CHUNK, 0, n_chunks)

        zeros16 = jnp.zeros((16,), jnp.float32)

        def zero_body(r, _):
            for g in range(D_FEAT // 16):
                gbuf[0, r, pl.ds(g * 16, 16)] = zeros16
            return 0
        lax.fori_loop(0, CHUNK, zero_body, 0)

        base_row = pl.multiple_of(s * ROWS_PER_TILE, 8)
        pltpu.sync_copy(gbuf.at[0], acc_sh.at[pl.ds(base_row, CHUNK)])
        pltpu.sync_copy(gbuf.at[0, pl.ds(0, ROWS_PER_TILE - CHUNK)],
                        acc_sh.at[pl.ds(base_row + CHUNK,
                                        ROWS_PER_TILE - CHUNK)])

        @pl.when(s == 0)
        def _():
            pltpu.sync_copy(gbuf.at[0, pl.ds(0, TRASH_ROWS)],
                            acc_sh.at[pl.ds(N_HALF, TRASH_ROWS)])
        plsc.subcore_barrier()

        def load_src(g):
            row0 = pl.multiple_of(idx_base + g * GROUP, 8)
            pltpu.sync_copy(src_hbm.at[pl.ds(row0, GROUP)], src_v)

        def load_dst(g):
            row0 = pl.multiple_of(idx_base + g * GROUP, 8)
            pltpu.sync_copy(dst_hbm.at[pl.ds(row0, GROUP)], dst_v)

        def start_gather(ci, b):
            pltpu.async_copy(feat_hbm.at[src_v.at[ci % GROUP]], gbuf.at[b],
                             gsem[b])
            e0 = pl.multiple_of(edge_base + ci * CHUNK, CHUNK)
            pltpu.async_copy(phys_hbm.at[pl.ds(e0, CHUNK)], pbuf.at[b],
                             gsem[b])

        def wait_gather(b):
            pltpu.make_async_copy(feat_hbm.at[pl.ds(0, CHUNK)], gbuf.at[b],
                                  gsem[b]).wait()
            pltpu.make_async_copy(phys_hbm.at[pl.ds(0, CHUNK)], pbuf.at[b],
                                  gsem[b]).wait()

        def wait_scatter(b):
            pltpu.make_async_copy(out_hbm.at[0, pl.ds(0, CHUNK)], gbuf.at[b],
                                  ssem[b]).wait()

        lanes = lax.iota(jnp.int32, 16)

        def scale(b, j):
            def scale_body(gi, _):
                for e in range(16):
                    row = gi * 16 + e
                    w = pbuf[b, row, :]
                    for m in (8, 4, 2, 1):
                        w = w + w.at[lanes ^ m].get(mode="promise_in_bounds")
                    for g in range(D_FEAT // 16):
                        sl = pl.ds(g * 16, 16)
                        gbuf[b, row, sl] = gbuf[b, row, sl] * w
                return 0
            lax.fori_loop(0, CHUNK // 16, scale_body, 0)

            r = j % GROUP
            for g8 in range(CHUNK // 16):
                dvec = dst_v[r, pl.ds(g8 * 16, 16)] - lo
                in_half = (dvec >= 0) & (dvec < N_HALF)
                trash = jnp.full((16,), N_HALF + g8 * 16, jnp.int32) + lanes
                adj_v[b, pl.ds(g8 * 16, 16)] = jnp.where(in_half, dvec, trash)

        load_src(0)
        load_dst(0)

        @pl.when(nv > 0)
        def _():
            start_gather(0, 0)

        def pair_body(t, _):
            for b in range(2):
                j = t * 2 + b
                g = j // GROUP

                @pl.when(j < nv)
                def _(b=b, j=j):
                    wait_gather(b)
                    scale(b, j)
                    pltpu.async_copy(gbuf.at[b], acc_sh.at[adj_v.at[b]],
                                     ssem[b], add=True)

                @pl.when((j % GROUP == GROUP - 1) & (g + 1 < n_groups))
                def _(g=g):
                    load_src(g + 1)
                    load_dst(g + 1)

                @pl.when(j + 1 < nv)
                def _(b=b, j=j):
                    @pl.when(j >= 1)
                    def _():
                        wait_scatter(1 - b)
                    start_gather(j + 1, 1 - b)
            return 0

        lax.fori_loop(0, n_chunks // 2, pair_body, 0)

        for b in range(2):
            wait_scatter(b)

        plsc.subcore_barrier()

        pltpu.sync_copy(acc_sh.at[pl.ds(base_row, ROWS_PER_TILE)],
                        out_hbm.at[c, pl.ds(base_row, ROWS_PER_TILE)])

    return k(features, src2d, dst2d, phys)


@jax.jit
def kernel(features, adjacency_list, physics_features):
    n_edges = adjacency_list.shape[1]
    align = GROUP * 128
    edges_per_tile = -(-n_edges // (NS * align)) * align
    e_pad = edges_per_tile * NS
    pad = e_pad - n_edges

    src = adjacency_list[0].astype(jnp.int32)
    dst = adjacency_list[1].astype(jnp.int32)
    if pad:
        src = jnp.concatenate([src, jnp.zeros((pad,), jnp.int32)])
        dst = jnp.concatenate([dst, jnp.zeros((pad,), jnp.int32)])

    src2d = src.reshape(e_pad // 128, 128)
    dst2d = dst.reshape(e_pad // 128, 128)
    phys = physics_features.astype(jnp.float32)

    halves = _sc_aggregate(features, src2d, dst2d, phys,
                           n_edges, edges_per_tile)
    return _assemble(halves)


def _assemble(halves):
    blk = 80

    def cat_k(p_ref, o_ref):
        o_ref[...] = p_ref[0]

    return pl.pallas_call(
        cat_k,
        grid=(N_NODES // blk,),
        in_specs=[pl.BlockSpec(
            (1, blk, D_FEAT),
            lambda i: (jnp.where(i < N_HALF // blk, 0, 1),
                       jnp.where(i < N_HALF // blk, i, i - N_HALF // blk),
                       0))],
        out_specs=pl.BlockSpec((blk, D_FEAT), lambda i: (i, 0)),
        out_shape=jax.ShapeDtypeStruct((N_NODES, D_FEAT), jnp.float32),
    )(halves)


# speedup vs baseline: 3.0931x; 1.0878x over previous
import functools

import jax
import jax.numpy as jnp
from jax import lax
from jax.experimental import pallas as pl
from jax.experimental.pallas import tpu as pltpu
from jax.experimental.pallas import tpu_sc as plsc

N_NODES = 10000
N_HALF = 5120
TRASH_ROWS = 128
ACC_ROWS = N_HALF + TRASH_ROWS
D_FEAT = 128
D_EDGE = 16

NC = 2
NS = 16

CHUNK = 128
GROUP = 16
ROWS_PER_TILE = N_HALF // NS


def _sc_aggregate(features, src2d, dst2d, phys, n_edges, edges_per_tile):
    n_chunks = edges_per_tile // CHUNK
    n_groups = n_chunks // GROUP

    mesh = plsc.VectorSubcoreMesh(core_axis_name="c", subcore_axis_name="s")

    @functools.partial(
        pl.kernel,
        mesh=mesh,
        out_type=jax.ShapeDtypeStruct((NC * N_HALF, D_FEAT), jnp.float32),
        scratch_types=[
            pltpu.VMEM((2, CHUNK, D_FEAT), jnp.float32),
            pltpu.VMEM((2, CHUNK, D_EDGE), jnp.float32),
            pltpu.VMEM((GROUP, 128), jnp.int32),
            pltpu.VMEM((GROUP, 128), jnp.int32),
            pltpu.VMEM((2, 128), jnp.int32),
            pltpu.SemaphoreType.DMA,
            pltpu.SemaphoreType.DMA,
            pltpu.SemaphoreType.DMA,
            pltpu.SemaphoreType.DMA,
            pltpu.VMEM_SHARED((ACC_ROWS, D_FEAT), jnp.float32),
        ],
    )
    def k(feat_hbm, src_hbm, dst_hbm, phys_hbm, out_hbm,
          gbuf, pbuf, src_v, dst_v, adj_v,
          gsem0, gsem1, ssem0, ssem1, acc_sh):
        gsem = (gsem0, gsem1)
        ssem = (ssem0, ssem1)
        c = lax.axis_index("c")
        s = lax.axis_index("s")
        lo = c * N_HALF
        edge_base = s * edges_per_tile
        idx_base = pl.multiple_of(edge_base // 128, 8)
        nv = jnp.clip((n_edges - edge_base) // CHUNK, 0, n_chunks)

        zeros16 = jnp.zeros((16,), jnp.float32)

        def zero_body(r, _):
            for g in range(D_FEAT // 16):
                gbuf[0, r, pl.ds(g * 16, 16)] = zeros16
            return 0
        lax.fori_loop(0, CHUNK, zero_body, 0)

        base_row = pl.multiple_of(s * ROWS_PER_TILE, 8)
        pltpu.sync_copy(gbuf.at[0], acc_sh.at[pl.ds(base_row, CHUNK)])
        pltpu.sync_copy(gbuf.at[0, pl.ds(0, ROWS_PER_TILE - CHUNK)],
                        acc_sh.at[pl.ds(base_row + CHUNK,
                                        ROWS_PER_TILE - CHUNK)])

        @pl.when(s == 0)
        def _():
            pltpu.sync_copy(gbuf.at[0, pl.ds(0, TRASH_ROWS)],
                            acc_sh.at[pl.ds(N_HALF, TRASH_ROWS)])
        plsc.subcore_barrier()

        def load_src(g):
            row0 = pl.multiple_of(idx_base + g * GROUP, 8)
            pltpu.sync_copy(src_hbm.at[pl.ds(row0, GROUP)], src_v)

        def load_dst(g):
            row0 = pl.multiple_of(idx_base + g * GROUP, 8)
            pltpu.sync_copy(dst_hbm.at[pl.ds(row0, GROUP)], dst_v)

        def start_gather(ci, b):
            pltpu.async_copy(feat_hbm.at[src_v.at[ci % GROUP]], gbuf.at[b],
                             gsem[b])
            e0 = pl.multiple_of(edge_base + ci * CHUNK, CHUNK)
            pltpu.async_copy(phys_hbm.at[pl.ds(e0, CHUNK)], pbuf.at[b],
                             gsem[b])

        def wait_gather(b):
            pltpu.make_async_copy(feat_hbm.at[pl.ds(0, CHUNK)], gbuf.at[b],
                                  gsem[b]).wait()
            pltpu.make_async_copy(phys_hbm.at[pl.ds(0, CHUNK)], pbuf.at[b],
                                  gsem[b]).wait()

        def wait_scatter(b):
            pltpu.make_async_copy(out_hbm.at[pl.ds(0, CHUNK)], gbuf.at[b],
                                  ssem[b]).wait()

        lanes = lax.iota(jnp.int32, 16)

        def scale(b, j):
            def scale_body(gi, _):
                for e in range(16):
                    row = gi * 16 + e
                    w = pbuf[b, row, :]
                    for m in (8, 4, 2, 1):
                        w = w + w.at[lanes ^ m].get(mode="promise_in_bounds")
                    for g in range(D_FEAT // 16):
                        sl = pl.ds(g * 16, 16)
                        gbuf[b, row, sl] = gbuf[b, row, sl] * w
                return 0
            lax.fori_loop(0, CHUNK // 16, scale_body, 0)

            r = j % GROUP
            for g8 in range(CHUNK // 16):
                dvec = dst_v[r, pl.ds(g8 * 16, 16)] - lo
                in_half = (dvec >= 0) & (dvec < N_HALF)
                trash = jnp.full((16,), N_HALF + g8 * 16, jnp.int32) + lanes
                adj_v[b, pl.ds(g8 * 16, 16)] = jnp.where(in_half, dvec, trash)

        load_src(0)
        load_dst(0)

        @pl.when(nv > 0)
        def _():
            start_gather(0, 0)

        def pair_body(t, _):
            for b in range(2):
                j = t * 2 + b
                g = j // GROUP

                @pl.when(j < nv)
                def _(b=b, j=j):
                    wait_gather(b)
                    scale(b, j)
                    pltpu.async_copy(gbuf.at[b], acc_sh.at[adj_v.at[b]],
                                     ssem[b], add=True)

                @pl.when((j % GROUP == GROUP - 1) & (g + 1 < n_groups))
                def _(g=g):
                    load_src(g + 1)
                    load_dst(g + 1)

                @pl.when(j + 1 < nv)
                def _(b=b, j=j):
                    @pl.when(j >= 1)
                    def _():
                        wait_scatter(1 - b)
                    start_gather(j + 1, 1 - b)
            return 0

        lax.fori_loop(0, n_chunks // 2, pair_body, 0)

        for b in range(2):
            wait_scatter(b)

        plsc.subcore_barrier()

        out_row = pl.multiple_of(c * N_HALF + base_row, 8)
        pltpu.sync_copy(acc_sh.at[pl.ds(base_row, ROWS_PER_TILE)],
                        out_hbm.at[pl.ds(out_row, ROWS_PER_TILE)])

    return k(features, src2d, dst2d, phys)


@jax.jit
def kernel(features, adjacency_list, physics_features):
    n_edges = adjacency_list.shape[1]
    align = GROUP * 128
    edges_per_tile = -(-n_edges // (NS * align)) * align
    e_pad = edges_per_tile * NS
    pad = e_pad - n_edges

    src = adjacency_list[0].astype(jnp.int32)
    dst = adjacency_list[1].astype(jnp.int32)
    if pad:
        src = jnp.concatenate([src, jnp.zeros((pad,), jnp.int32)])
        dst = jnp.concatenate([dst, jnp.zeros((pad,), jnp.int32)])

    src2d = src.reshape(e_pad // 128, 128)
    dst2d = dst.reshape(e_pad // 128, 128)
    phys = physics_features.astype(jnp.float32)

    halves = _sc_aggregate(features, src2d, dst2d, phys,
                           n_edges, edges_per_tile)
    return _assemble(halves)


def _assemble(stacked):
    blk = 1000

    def slice_k(p_ref, o_ref):
        o_ref[...] = p_ref[...]

    return pl.pallas_call(
        slice_k,
        grid=(N_NODES // blk,),
        in_specs=[pl.BlockSpec((blk, D_FEAT), lambda i: (i, 0))],
        out_specs=pl.BlockSpec((blk, D_FEAT), lambda i: (i, 0)),
        out_shape=jax.ShapeDtypeStruct((N_NODES, D_FEAT), jnp.float32),
    )(stacked)


# speedup vs baseline: 5.0125x; 1.6205x over previous
import functools

import jax
import jax.numpy as jnp
from jax import lax
from jax.experimental import pallas as pl
from jax.experimental.pallas import tpu as pltpu
from jax.experimental.pallas import tpu_sc as plsc

N_NODES = 10000
N_PAD = 10112
D_FEAT = 128
D_EDGE = 16

NC = 2
NS = 16
NW = NC * NS

CHUNK = 128
GROUP = 8
ROWS_PER_TILE = N_PAD // NS


def _sc_aggregate(features, src2d, dst2d, phys, n_edges, edges_per_tile):
    n_chunks = edges_per_tile // CHUNK
    n_groups = n_chunks // GROUP

    mesh = plsc.VectorSubcoreMesh(core_axis_name="c", subcore_axis_name="s")

    @functools.partial(
        pl.kernel,
        mesh=mesh,
        out_type=jax.ShapeDtypeStruct((NC, N_PAD, D_FEAT), jnp.float32),
        scratch_types=[
            pltpu.VMEM((CHUNK, D_FEAT), jnp.float32),
            pltpu.VMEM((CHUNK, D_EDGE), jnp.float32),
            pltpu.VMEM((GROUP, 128), jnp.int32),
            pltpu.VMEM((GROUP, 128), jnp.int32),
            pltpu.SemaphoreType.DMA,
            pltpu.VMEM_SHARED((N_PAD, D_FEAT), jnp.float32),
        ],
    )
    def k(feat_hbm, src_hbm, dst_hbm, phys_hbm, out_hbm,
          gbuf, pbuf, src_v, dst_v, gsem, acc_sh):
        c = lax.axis_index("c")
        s = lax.axis_index("s")
        wid = s * NC + c
        edge_base = wid * edges_per_tile
        idx_base = pl.multiple_of(edge_base // 128, 8)
        nv = jnp.clip((n_edges - edge_base) // CHUNK, 0, n_chunks)

        zeros16 = jnp.zeros((16,), jnp.float32)

        def zero_body(r, _):
            for g in range(D_FEAT // 16):
                gbuf[r, pl.ds(g * 16, 16)] = zeros16
            return 0
        lax.fori_loop(0, CHUNK, zero_body, 0)

        base_row = pl.multiple_of(s * ROWS_PER_TILE, 8)
        done = 0
        while done < ROWS_PER_TILE:
            n = min(CHUNK, ROWS_PER_TILE - done)
            pltpu.sync_copy(gbuf.at[pl.ds(0, n)],
                            acc_sh.at[pl.ds(base_row + done, n)])
            done += n
        plsc.subcore_barrier()

        def load_idx(g):
            row0 = pl.multiple_of(idx_base + g * GROUP, 8)
            pltpu.sync_copy(src_hbm.at[pl.ds(row0, GROUP)], src_v)
            pltpu.sync_copy(dst_hbm.at[pl.ds(row0, GROUP)], dst_v)

        lanes = lax.iota(jnp.int32, 16)

        def scale():
            def scale_body(gi, _):
                for e in range(16):
                    row = gi * 16 + e
                    w = pbuf[row, :]
                    for m in (8, 4, 2, 1):
                        w = w + w.at[lanes ^ m].get(mode="promise_in_bounds")
                    for g in range(D_FEAT // 16):
                        sl = pl.ds(g * 16, 16)
                        gbuf[row, sl] = gbuf[row, sl] * w
                return 0
            lax.fori_loop(0, CHUNK // 16, scale_body, 0)

        def chunk_body(j, _):
            @pl.when(j % GROUP == 0)
            def _():
                load_idx(j // GROUP)

            @pl.when(j < nv)
            def _():
                r = j % GROUP
                pltpu.async_copy(feat_hbm.at[src_v.at[r]], gbuf, gsem)
                e0 = pl.multiple_of(edge_base + j * CHUNK, CHUNK)
                cp = pltpu.async_copy(phys_hbm.at[pl.ds(e0, CHUNK)], pbuf,
                                      gsem)
                pltpu.make_async_copy(feat_hbm.at[pl.ds(0, CHUNK)], gbuf,
                                      gsem).wait()
                cp.wait()
                scale()
                pltpu.sync_copy(gbuf, acc_sh.at[dst_v.at[r]], add=True)
            return 0

        lax.fori_loop(0, n_chunks, chunk_body, 0)
        plsc.subcore_barrier()

        pltpu.sync_copy(acc_sh.at[pl.ds(base_row, ROWS_PER_TILE)],
                        out_hbm.at[c, pl.ds(base_row, ROWS_PER_TILE)])

    return k(features, src2d, dst2d, phys)


def _combine_partials(partials):
    blk = 1000

    def add_k(p_ref, o_ref):
        o_ref[...] = p_ref[0] + p_ref[1]

    return pl.pallas_call(
        add_k,
        grid=(N_NODES // blk,),
        in_specs=[pl.BlockSpec((NC, blk, D_FEAT), lambda i: (0, i, 0))],
        out_specs=pl.BlockSpec((blk, D_FEAT), lambda i: (i, 0)),
        out_shape=jax.ShapeDtypeStruct((N_NODES, D_FEAT), jnp.float32),
    )(partials)


@jax.jit
def kernel(features, adjacency_list, physics_features):
    n_edges = adjacency_list.shape[1]
    align = GROUP * 128
    edges_per_tile = -(-n_edges // (NW * align)) * align
    e_pad = edges_per_tile * NW
    pad = e_pad - n_edges

    src = adjacency_list[0].astype(jnp.int32)
    dst = adjacency_list[1].astype(jnp.int32)
    if pad:
        src = jnp.concatenate([src, jnp.zeros((pad,), jnp.int32)])
        dst = jnp.concatenate([dst, jnp.zeros((pad,), jnp.int32)])

    src2d = src.reshape(e_pad // 128, 128)
    dst2d = dst.reshape(e_pad // 128, 128)
    phys = physics_features.astype(jnp.float32)

    partials = _sc_aggregate(features, src2d, dst2d, phys,
                             n_edges, edges_per_tile)
    return _combine_partials(partials)
